# Initial kernel scaffold; baseline (speedup 1.0000x reference)
import jax
import jax.numpy as jnp
from jax.experimental import pallas as pl


def kernel(embeddings, hit_score, hit_particle_id, signal_edges, random_edges):
    raise NotImplementedError("write your pallas kernel here")


# speedup vs baseline: 3.0842x; 3.0842x over previous
import functools

import jax
import jax.numpy as jnp
from jax.experimental import pallas as pl
from jax.experimental.pallas import tpu as pltpu

N = 8192
D = 64
E_SIG = 32768
E_RND = 65536
E_ALL = E_SIG + E_RND
K1 = 17
MARGIN = 1.0
R_MAX = 100.0

BR = 256
NSTRIPES = N // BR
BE = 4096
NEBLK = E_ALL // BE


def _knn_body(emb_row_ref, emb_all_ref, pid_row_ref, pid_col_ref, out_ref,
              stripe_ref, pm_ref):
    i = pl.program_id(0)
    a = emb_row_ref[...]
    b = emb_all_ref[...]
    dots = jax.lax.dot_general(
        a, b, (((1,), (1,)), ((), ())),
        preferred_element_type=jnp.float32,
        precision=jax.lax.Precision.HIGHEST)
    sq_r = jnp.sum(a * a, axis=1, keepdims=True)
    sq_c = jax.lax.dot_general(
        jnp.ones((8, D), jnp.float32), b * b, (((1,), (1,)), ((), ())),
        preferred_element_type=jnp.float32,
        precision=jax.lax.Precision.HIGHEST)[:1]
    stripe_ref[...] = sq_r + sq_c - 2.0 * dots

    pid_r = pid_row_ref[...]
    pid_c = pid_col_ref[...]
    pm_ref[...] = jnp.where((pid_r == pid_c) & (pid_r != 0), 1.0, 0.0)

    colio = jax.lax.broadcasted_iota(jnp.int32, (BR, N), 1)
    rowid = i * BR + jax.lax.broadcasted_iota(jnp.int32, (BR, 1), 0)

    def body(_, carry):
        num, den = carry
        s = stripe_ref[...]
        vmin = jnp.min(s, axis=1, keepdims=True)
        amin = jnp.min(jnp.where(s == vmin, colio, jnp.int32(N)),
                       axis=1, keepdims=True)
        eq2 = colio == amin
        stripe_ref[...] = jnp.where(eq2, jnp.float32(3.0e38), s)
        pm = jnp.sum(jnp.where(eq2, pm_ref[...], 0.0),
                     axis=1, keepdims=True)
        d = jnp.sqrt(jnp.maximum(vmin, 0.0) + 1e-12)
        valid = (amin != rowid) & (d <= R_MAX)
        l = jnp.where(pm > 0.5, d, jnp.maximum(0.0, MARGIN - d))
        mf = jnp.where(valid, 1.0, 0.0)
        return num + l * mf, den + mf

    zero = jnp.zeros((BR, 1), jnp.float32)
    num, den = jax.lax.fori_loop(0, K1, body, (zero, zero))
    lane = jax.lax.broadcasted_iota(jnp.int32, (1, 1, 128), 2)
    out_ref[...] = jnp.where(lane == 0, jnp.sum(num),
                             jnp.where(lane == 1, jnp.sum(den), 0.0))


def _edge_body(es_ref, ed_ref, y_ref, wsig_ref, hs_ref, pid_ref, out_ref):
    i = pl.program_id(0)
    diff = es_ref[...] - ed_ref[...]
    d = jnp.sqrt(jnp.sum(diff * diff, axis=1, keepdims=True) + 1e-12)
    y = y_ref[...]
    l = jnp.where(y > 0.5, d, jnp.maximum(0.0, MARGIN - d))
    w = wsig_ref[...]
    ssum = jnp.sum(l * w)
    rsum = jnp.sum(l * (1.0 - w))

    def beta():
        x = hs_ref[...]
        t = jnp.where(pid_ref[...] != 0, 1.0, 0.0)
        bce = (jnp.maximum(x, 0.0) - x * t
               + jnp.log1p(jnp.exp(-jnp.abs(x))))
        return jnp.sum(bce)

    bsum = jax.lax.cond(i == 0, beta, lambda: jnp.float32(0.0))
    lane = jax.lax.broadcasted_iota(jnp.int32, (1, 1, 128), 2)
    out_ref[...] = jnp.where(lane == 0, ssum,
                             jnp.where(lane == 1, rsum,
                                       jnp.where(lane == 2, bsum, 0.0)))


@functools.partial(jax.jit, static_argnames=())
def kernel(embeddings, hit_score, hit_particle_id, signal_edges, random_edges):
    emb = embeddings.astype(jnp.float32)
    pid = hit_particle_id.astype(jnp.int32)
    pid_row = pid.reshape(N, 1)
    pid_col = pid.reshape(1, N)

    knn_part = pl.pallas_call(
        _knn_body,
        grid=(NSTRIPES,),
        in_specs=[
            pl.BlockSpec((BR, D), lambda i: (i, 0)),
            pl.BlockSpec((N, D), lambda i: (0, 0)),
            pl.BlockSpec((BR, 1), lambda i: (i, 0)),
            pl.BlockSpec((1, N), lambda i: (0, 0)),
        ],
        out_specs=pl.BlockSpec((1, 1, 128), lambda i: (i, 0, 0)),
        out_shape=jax.ShapeDtypeStruct((NSTRIPES, 1, 128), jnp.float32),
        scratch_shapes=[
            pltpu.VMEM((BR, N), jnp.float32),
            pltpu.VMEM((BR, N), jnp.float32),
        ],
    )(emb, emb, pid_row, pid_col)

    src = jnp.concatenate([signal_edges[0], random_edges[0]])
    dst = jnp.concatenate([signal_edges[1], random_edges[1]])
    es = emb[src]
    ed = emb[dst]
    ps = pid[src]
    pd = pid[dst]
    y = jnp.where((ps == pd) & (ps != 0), 1.0, 0.0).reshape(E_ALL, 1)
    wsig = (jnp.arange(E_ALL) < E_SIG).astype(jnp.float32).reshape(E_ALL, 1)

    edge_part = pl.pallas_call(
        _edge_body,
        grid=(NEBLK,),
        in_specs=[
            pl.BlockSpec((BE, D), lambda i: (i, 0)),
            pl.BlockSpec((BE, D), lambda i: (i, 0)),
            pl.BlockSpec((BE, 1), lambda i: (i, 0)),
            pl.BlockSpec((BE, 1), lambda i: (i, 0)),
            pl.BlockSpec((N, 1), lambda i: (0, 0)),
            pl.BlockSpec((N, 1), lambda i: (0, 0)),
        ],
        out_specs=pl.BlockSpec((1, 1, 128), lambda i: (i, 0, 0)),
        out_shape=jax.ShapeDtypeStruct((NEBLK, 1, 128), jnp.float32),
    )(es, ed, y, wsig, hit_score.astype(jnp.float32).reshape(N, 1), pid_row)

    knn_num = jnp.sum(knn_part[:, 0, 0])
    knn_den = jnp.sum(knn_part[:, 0, 1])
    knn_loss = knn_num / jnp.maximum(knn_den, 1.0)
    sig_sum = jnp.sum(edge_part[:, 0, 0])
    rnd_sum = jnp.sum(edge_part[:, 0, 1])
    beta_sum = jnp.sum(edge_part[:, 0, 2])
    signal_loss = sig_sum / float(E_SIG)
    random_loss = rnd_sum / float(E_RND)
    beta_loss = beta_sum / float(N)
    total = signal_loss + knn_loss + random_loss + beta_loss
    return jnp.stack([total, signal_loss, knn_loss, random_loss, beta_loss])
